# baseline (speedup 1.0000x reference)
import functools

import jax
import jax.numpy as jnp
from jax import lax
from jax.experimental import pallas as pl
from jax.experimental.pallas import tpu as pltpu
from jax.experimental.pallas import tpu_sc as plsc

_NC = 2
_NS = 16
_NW = _NC * _NS
_LANES = 16
_NSLOT = 4


def _make_sc_kernel(B, S, D):
    rows_per_w = S // _NW
    CH = 16
    n_chunks = rows_per_w // CH
    vregs_per_row = D // _LANES

    mesh = plsc.VectorSubcoreMesh(core_axis_name="c", subcore_axis_name="s")

    @functools.partial(
        pl.kernel,
        out_type=jax.ShapeDtypeStruct((B, S, D), jnp.float32),
        mesh=mesh,
        compiler_params=pltpu.CompilerParams(use_tc_tiling_on_sc=True),
        scratch_types=[
            pltpu.VMEM((_NSLOT, B, CH, D), jnp.float32),
            pltpu.VMEM((_NSLOT, CH, D), jnp.float32),
            pltpu.SemaphoreType.DMA((_NSLOT,)),
            pltpu.SemaphoreType.DMA((_NSLOT,)),
        ],
    )
    def sc_kernel(in_hbm, pos_hbm, out_hbm, io_b, pos_b, sin, sout):
        wid = lax.axis_index("s") * _NC + lax.axis_index("c")
        row_base = wid * rows_per_w

        def in_descs(k, t):
            r0 = row_base + k * CH
            return [
                pltpu.make_async_copy(pos_hbm.at[pl.ds(r0, CH), :], pos_b.at[t], sin.at[t]),
                pltpu.make_async_copy(
                    in_hbm.at[:, pl.ds(r0, CH), :], io_b.at[t], sin.at[t]
                ),
            ]

        def out_descs(k, t):
            r0 = row_base + k * CH
            return [
                pltpu.make_async_copy(
                    io_b.at[t], out_hbm.at[:, pl.ds(r0, CH), :], sout.at[t]
                )
            ]

        def start_in(k, t):
            for d in in_descs(k, t):
                d.start()

        def compute(t):
            @plsc.parallel_loop(0, CH)
            def _(row):
                for c in range(vregs_per_row):
                    cs = pl.ds(c * _LANES, _LANES)
                    po = pos_b[t, row, cs]
                    for b in range(B):
                        plsc.addupdate(io_b.at[t, b, row, cs], po)

        start_in(0, 0)
        start_in(1, 1)
        start_in(2, 2)

        @pl.loop(0, n_chunks)
        def _(k):
            t = lax.rem(k, _NSLOT)
            for d in in_descs(k, t):
                d.wait()

            @pl.when(k >= 2)
            def _():
                for d in out_descs(k - 2, lax.rem(k - 2, _NSLOT)):
                    d.wait()

            @pl.when((k >= 1) & (k + 2 < n_chunks))
            def _():
                start_in(k + 2, lax.rem(k + 2, _NSLOT))

            compute(t)
            for d in out_descs(k, t):
                d.start()

        for k in (n_chunks - 2, n_chunks - 1):
            for d in out_descs(k, k % _NSLOT):
                d.wait()

    return sc_kernel


def kernel(input_embeddings, pos_table):
    B, S, D = input_embeddings.shape
    return _make_sc_kernel(B, S, D)(input_embeddings, pos_table[:S])


# speedup vs baseline: 1.0284x; 1.0284x over previous
import functools

import jax
import jax.numpy as jnp
from jax import lax
from jax.experimental import pallas as pl
from jax.experimental.pallas import tpu as pltpu
from jax.experimental.pallas import tpu_sc as plsc

_NC = 2
_NS = 16
_NW = _NC * _NS
_LANES = 16
_NSLOT = 4


def _make_sc_kernel(B, S, D):
    rows_per_w = S // _NW
    CH = 16
    n_chunks = rows_per_w // CH
    vregs_per_row = D // _LANES

    mesh = plsc.VectorSubcoreMesh(core_axis_name="c", subcore_axis_name="s")

    @functools.partial(
        pl.kernel,
        out_type=jax.ShapeDtypeStruct((B, S, D), jnp.float32),
        mesh=mesh,
        compiler_params=pltpu.CompilerParams(use_tc_tiling_on_sc=True),
        scratch_types=[
            pltpu.VMEM((_NSLOT, B, CH, D), jnp.float32),
            pltpu.VMEM((_NSLOT, CH, D), jnp.float32),
            pltpu.SemaphoreType.DMA((_NSLOT,)),
            pltpu.SemaphoreType.DMA((_NSLOT,)),
        ],
    )
    def sc_kernel(in_hbm, pos_hbm, out_hbm, io_b, pos_b, sin, sout):
        wid = lax.axis_index("s") * _NC + lax.axis_index("c")
        row_base = wid * rows_per_w

        def in_descs(k, t):
            r0 = row_base + k * CH
            return [
                pltpu.make_async_copy(pos_hbm.at[pl.ds(r0, CH), :], pos_b.at[t], sin.at[t]),
                pltpu.make_async_copy(
                    in_hbm.at[:, pl.ds(r0, CH), :], io_b.at[t], sin.at[t]
                ),
            ]

        def out_descs(k, t):
            r0 = row_base + k * CH
            return [
                pltpu.make_async_copy(
                    io_b.at[t], out_hbm.at[:, pl.ds(r0, CH), :], sout.at[t]
                )
            ]

        def start_in(k, t):
            for d in in_descs(k, t):
                d.start()

        def compute(t):
            @plsc.parallel_loop(0, CH)
            def _(row):
                for c in range(vregs_per_row):
                    cs = pl.ds(c * _LANES, _LANES)
                    po = pos_b[t, row, cs]
                    for b in range(B):
                        plsc.addupdate(io_b.at[t, b, row, cs], po)

        start_in(0, 0)
        start_in(1, 1)

        @pl.loop(0, n_chunks)
        def _(k):
            t = lax.rem(k, _NSLOT)
            for d in in_descs(k, t):
                d.wait()

            @pl.when(k >= 2)
            def _():
                for d in out_descs(k - 2, lax.rem(k - 2, _NSLOT)):
                    d.wait()

            @pl.when(k + 2 < n_chunks)
            def _():
                start_in(k + 2, lax.rem(k + 2, _NSLOT))

            compute(t)
            for d in out_descs(k, t):
                d.start()

        for k in (n_chunks - 2, n_chunks - 1):
            for d in out_descs(k, k % _NSLOT):
                d.wait()

    return sc_kernel


def kernel(input_embeddings, pos_table):
    B, S, D = input_embeddings.shape
    return _make_sc_kernel(B, S, D)(input_embeddings, pos_table[:S])
